# baseline (speedup 1.0000x reference)
import jax
import jax.numpy as jnp
from jax import lax
from jax.experimental import pallas as pl
from jax.experimental.pallas import tpu as pltpu

_L = 64
_KH, _KW = 4, 7
_HALF = _KW // 2
_BB = 64


def _fmc_kernel(scal_ref, inp_ref, cache_ref, k_ref, bias_ref, y_ref, cout_ref):
    index_w = scal_ref[0]
    iw_in = scal_ref[1]
    do_update = scal_ref[2]
    do_shift = scal_ref[3]

    @pl.when(do_shift == 0)
    def _():
        cout_ref[...] = cache_ref[...]

        @pl.when(do_update == 1)
        def _():
            cout_ref[:, _KH - 1, pl.ds(iw_in, 1), :] = inp_ref[...][:, None, :]

    @pl.when(do_shift == 1)
    def _():
        cout_ref[:, 0 : _KH - 2, :, :] = cache_ref[:, 1 : _KH - 1, :, :]
        r3 = cache_ref[:, _KH - 1, :, :]
        colmask = lax.broadcasted_iota(jnp.int32, (1, _L, 1), 1) == _L - 1
        r3 = jnp.where(colmask, inp_ref[...][:, None, :], r3)
        cout_ref[:, _KH - 2, :, :] = r3
        cout_ref[:, _KH - 1, :, :] = jnp.zeros_like(r3)

    acc = jnp.zeros((inp_ref.shape[0], k_ref.shape[3]), jnp.float32)
    for h in range(_KH):
        wmax = _HALF if h == _KH - 1 else _KW
        for w in range(wmax):
            col = index_w - _HALF + w
            valid = jnp.where((col >= 0) & (col < _L), 1.0, 0.0)
            ccol = jnp.clip(col, 0, _L - 1)
            x = cout_ref[:, h, pl.ds(ccol, 1), :][:, 0, :]
            acc = acc + jnp.dot(
                x * valid, k_ref[h, w], preferred_element_type=jnp.float32
            )
    y_ref[...] = acc + bias_ref[...]


def kernel(inputs, cache, kernel, bias, index):
    batch, in_f = inputs.shape
    out_f = kernel.shape[3]
    index = jnp.asarray(index, jnp.int32)
    index_w = index % _L
    iw_in = (index - 1) % _L
    do_update = (index >= 1).astype(jnp.int32)
    do_shift = ((index >= 1) & (index_w == 0)).astype(jnp.int32)
    scalars = jnp.stack([index_w, iw_in, do_update, do_shift])

    nb = batch // _BB
    y, cache_out = pl.pallas_call(
        _fmc_kernel,
        grid=(nb,),
        in_specs=[
            pl.BlockSpec(memory_space=pltpu.SMEM),
            pl.BlockSpec((_BB, in_f), lambda i: (i, 0)),
            pl.BlockSpec((_BB, _KH, _L, in_f), lambda i: (i, 0, 0, 0)),
            pl.BlockSpec((_KH, _KW, in_f, out_f), lambda i: (0, 0, 0, 0)),
            pl.BlockSpec((1, out_f), lambda i: (0, 0)),
        ],
        out_specs=[
            pl.BlockSpec((_BB, out_f), lambda i: (i, 0)),
            pl.BlockSpec((_BB, _KH, _L, in_f), lambda i: (i, 0, 0, 0)),
        ],
        out_shape=[
            jax.ShapeDtypeStruct((batch, out_f), jnp.float32),
            jax.ShapeDtypeStruct(cache.shape, jnp.float32),
        ],
        compiler_params=pltpu.CompilerParams(
            dimension_semantics=("parallel",),
        ),
    )(scalars, inputs, cache, kernel, bias.reshape(1, out_f))
    return y, cache_out
